# baseline (speedup 1.0000x reference)
import functools

import jax
import jax.numpy as jnp
from jax import lax
from jax.experimental import pallas as pl
from jax.experimental.pallas import tpu as pltpu
from jax.experimental.pallas import tpu_sc as plsc

N = 10000
E = 320000
G = 64
NC, NS = 2, 16
NW = NC * NS
EPW = E // NW
CH = 128
NFULL = EPW // CH
TAIL = EPW - NFULL * CH
RPT = 624
RPT_LAST = N - (NS - 1) * RPT


def _sc_scatter(F):
    mesh = plsc.VectorSubcoreMesh(core_axis_name="c", subcore_axis_name="s",
                                  num_cores=NC, num_subcores=NS)

    @functools.partial(
        pl.kernel,
        out_type=jax.ShapeDtypeStruct((NC, N, F), jnp.float32),
        mesh=mesh,
        scratch_types=[
            pltpu.VMEM((CH,), jnp.int32),
            pltpu.VMEM((CH,), jnp.int32),
            pltpu.VMEM((CH, F), jnp.float32),
            pltpu.VMEM((TAIL,), jnp.int32),
            pltpu.VMEM((TAIL,), jnp.int32),
            pltpu.VMEM((TAIL, F), jnp.float32),
            pltpu.VMEM_SHARED((N, F), jnp.float32),
            pltpu.SemaphoreType.DMA,
        ],
        compiler_params=pltpu.CompilerParams(use_tc_tiling_on_sc=False),
    )
    def scatter_kernel(h_hbm, src_hbm, dst_hbm, zeros_hbm, out_hbm,
                       sidx, didx, rows, sidx_t, didx_t, rows_t, acc, sem):
        c = lax.axis_index("c")
        s = lax.axis_index("s")
        start = s * RPT

        @pl.when(s < NS - 1)
        def _():
            pltpu.sync_copy(zeros_hbm.at[pl.ds(0, RPT)],
                            acc.at[pl.ds(start, RPT)])

        @pl.when(s == NS - 1)
        def _():
            pltpu.sync_copy(zeros_hbm, acc.at[pl.ds(start, RPT_LAST)])

        plsc.subcore_barrier()
        base = (c * NS + s) * EPW

        def body(i, carry):
            off = base + i * CH
            pltpu.sync_copy(src_hbm.at[pl.ds(off, CH)], sidx)
            pltpu.sync_copy(dst_hbm.at[pl.ds(off, CH)], didx)
            pltpu.async_copy(h_hbm.at[sidx], rows, sem).wait()
            pltpu.sync_copy(rows, acc.at[didx], add=True)
            return carry

        lax.fori_loop(0, NFULL, body, 0)
        toff = base + NFULL * CH
        pltpu.sync_copy(src_hbm.at[pl.ds(toff, TAIL)], sidx_t)
        pltpu.sync_copy(dst_hbm.at[pl.ds(toff, TAIL)], didx_t)
        pltpu.async_copy(h_hbm.at[sidx_t], rows_t, sem).wait()
        pltpu.sync_copy(rows_t, acc.at[didx_t], add=True)
        plsc.subcore_barrier()

        @pl.when(s < NS - 1)
        def _():
            pltpu.sync_copy(acc.at[pl.ds(start, RPT)],
                            out_hbm.at[c, pl.ds(start, RPT)])

        @pl.when(s == NS - 1)
        def _():
            pltpu.sync_copy(acc.at[pl.ds(start, RPT_LAST)],
                            out_hbm.at[c, pl.ds(start, RPT_LAST)])

    return scatter_kernel


_SCAT64 = _sc_scatter(64)
_SCAT128 = _sc_scatter(128)


def _leaky(x):
    return jnp.where(x >= 0, x, 0.01 * x)


_BM = 1000
_BC = 2000


def _make_combine(F, n_extra, bias, act):
    def body(*refs):
        refs = list(refs)
        o_ref = refs.pop()
        b_ref = refs.pop() if bias else None
        p_ref = refs.pop(0)
        t = p_ref[0] + p_ref[1]
        for r in refs:
            t = t + r[...]
        if b_ref is not None:
            t = t + b_ref[...]
        if act:
            t = _leaky(t)
        o_ref[...] = t

    in_specs = [pl.BlockSpec((2, _BC, F), lambda i: (0, i, 0))]
    in_specs += [pl.BlockSpec((_BC, F), lambda i: (i, 0))] * n_extra
    if bias:
        in_specs.append(pl.BlockSpec((1, F), lambda i: (0, 0)))
    return pl.pallas_call(
        body,
        grid=(N // _BC,),
        in_specs=in_specs,
        out_specs=pl.BlockSpec((_BC, F), lambda i: (i, 0)),
        out_shape=jax.ShapeDtypeStruct((N, F), jnp.float32),
    )


_COMB64 = _make_combine(64, 0, False, False)
_COMB128 = _make_combine(128, 0, False, False)


def _make_mm4(fin, fout, act):
    def body(h0, h1, h2, h3, w_ref, b_ref, o_ref):
        acc = jnp.dot(h0[...], w_ref[0], preferred_element_type=jnp.float32)
        for k, h in enumerate((h1, h2, h3), start=1):
            acc = acc + jnp.dot(h[...], w_ref[k],
                                preferred_element_type=jnp.float32)
        acc = acc + b_ref[...]
        if act:
            acc = _leaky(acc)
        o_ref[...] = acc

    return pl.pallas_call(
        body,
        grid=(N // _BM,),
        in_specs=[pl.BlockSpec((_BM, fin), lambda i: (i, 0))] * 4
        + [pl.BlockSpec((4, fin, fout), lambda i: (0, 0, 0)),
           pl.BlockSpec((1, fout), lambda i: (0, 0))],
        out_specs=pl.BlockSpec((_BM, fout), lambda i: (i, 0)),
        out_shape=jax.ShapeDtypeStruct((N, fout), jnp.float32),
    )


_MM4_L1 = _make_mm4(128, 64, True)
_MM4_L2 = _make_mm4(64, 128, True)
_MM4_L3 = _make_mm4(128, 256, False)

_BP = 1000


def _pool_body(b_ref, h_ref, o_ref):
    @pl.when(pl.program_id(0) == 0)
    def _init():
        o_ref[...] = jnp.full((G, 256), -jnp.inf, jnp.float32)

    bi = b_ref[0]
    h = h_ref[...]
    lo = jnp.min(bi)
    hi = jnp.max(bi)

    def gbody(g, carry):
        m = bi == g
        v = jnp.max(jnp.where(m, h, -jnp.inf), axis=0, keepdims=True)
        o_ref[pl.ds(g, 1), :] = jnp.maximum(o_ref[pl.ds(g, 1), :], v)
        return carry

    lax.fori_loop(lo, hi + 1, gbody, 0)


_POOL = pl.pallas_call(
    _pool_body,
    grid=(N // _BP,),
    in_specs=[pl.BlockSpec((1, _BP, 1), lambda i: (i, 0, 0)),
              pl.BlockSpec((_BP, 256), lambda i: (i, 0))],
    out_specs=pl.BlockSpec((G, 256), lambda i: (0, 0)),
    out_shape=jax.ShapeDtypeStruct((G, 256), jnp.float32),
)


def _mlp_body(g_ref, w1, b1, w2, b2, w3, b3, o_ref):
    g = g_ref[...]
    g = jnp.where(jnp.isfinite(g), g, 0.0)
    h = jnp.maximum(
        jnp.dot(g, w1[...], preferred_element_type=jnp.float32) + b1[...], 0.0)
    h = jnp.maximum(
        jnp.dot(h, w2[...], preferred_element_type=jnp.float32) + b2[...], 0.0)
    o_ref[...] = jnp.dot(h, w3[...], preferred_element_type=jnp.float32) + b3[...]


_MLP = pl.pallas_call(
    _mlp_body,
    out_shape=jax.ShapeDtypeStruct((G, 4), jnp.float32),
)


def kernel(x, edge_index, batch, W1, b1, W2, b2, W3, b3,
           Wc1, bc1, Wc2, bc2, Wc3, bc3):
    src = edge_index[0]
    dst = edge_index[1]
    z64 = jnp.zeros((RPT_LAST, 64), jnp.float32)
    z128 = jnp.zeros((RPT_LAST, 128), jnp.float32)

    p = _SCAT128(x, src, dst, z128)
    a1 = _COMB128(p)
    p = _SCAT128(a1, src, dst, z128)
    a2 = _COMB128(p)
    p = _SCAT128(a2, src, dst, z128)
    a3 = _COMB128(p)
    h1 = _MM4_L1(x, a1, a2, a3, W1, b1.reshape(1, 64))

    p = _SCAT64(h1, src, dst, z64)
    h2 = _COMB64(p)
    p = _SCAT64(h2, src, dst, z64)
    h3 = _COMB64(p)
    p = _SCAT64(h3, src, dst, z64)
    h4 = _COMB64(p)
    z = _MM4_L2(h1, h2, h3, h4, W2, b2.reshape(1, 128))

    p = _SCAT128(z, src, dst, z128)
    g2 = _COMB128(p)
    p = _SCAT128(g2, src, dst, z128)
    g3 = _COMB128(p)
    p = _SCAT128(g3, src, dst, z128)
    g4 = _COMB128(p)
    hfin = _MM4_L3(z, g2, g3, g4, W3, b3.reshape(1, 256))

    gpool = _POOL(batch.reshape(N // _BP, _BP, 1), hfin)
    return _MLP(gpool, Wc1, bc1.reshape(1, 1024), Wc2, bc2.reshape(1, 512),
                Wc3, bc3.reshape(1, 4))


# speedup vs baseline: 2.1159x; 2.1159x over previous
import functools

import jax
import jax.numpy as jnp
from jax import lax
from jax.experimental import pallas as pl
from jax.experimental.pallas import tpu as pltpu
from jax.experimental.pallas import tpu_sc as plsc

N = 10000
E = 320000
G = 64
NC, NS = 2, 16
NW = NC * NS
EPW = E // NW
CH = 128
NFULL = EPW // CH
TAIL = EPW - NFULL * CH
RPT = 624
RPT_LAST = N - (NS - 1) * RPT


def _sc_scatter(F):
    mesh = plsc.VectorSubcoreMesh(core_axis_name="c", subcore_axis_name="s",
                                  num_cores=NC, num_subcores=NS)

    @functools.partial(
        pl.kernel,
        out_type=jax.ShapeDtypeStruct((NC, N, F), jnp.float32),
        mesh=mesh,
        scratch_types=[
            [pltpu.VMEM((CH,), jnp.int32)] * 3,
            [pltpu.VMEM((CH,), jnp.int32)] * 3,
            [pltpu.VMEM((CH, F), jnp.float32)] * 2,
            pltpu.VMEM((TAIL,), jnp.int32),
            pltpu.VMEM((TAIL,), jnp.int32),
            pltpu.VMEM((TAIL, F), jnp.float32),
            pltpu.VMEM_SHARED((N, F), jnp.float32),
            pltpu.SemaphoreType.DMA,
            pltpu.SemaphoreType.DMA,
        ],
        compiler_params=pltpu.CompilerParams(use_tc_tiling_on_sc=False),
    )
    def scatter_kernel(h_hbm, src_hbm, dst_hbm, zeros_hbm, out_hbm,
                       sidx, didx, rows, sidx_t, didx_t, rows_t, acc,
                       isem, gsem):
        c = lax.axis_index("c")
        s = lax.axis_index("s")
        start = s * RPT

        @pl.when(s < NS - 1)
        def _():
            pltpu.sync_copy(zeros_hbm.at[pl.ds(0, RPT)],
                            acc.at[pl.ds(start, RPT)])

        @pl.when(s == NS - 1)
        def _():
            pltpu.sync_copy(zeros_hbm, acc.at[pl.ds(start, RPT_LAST)])

        plsc.subcore_barrier()
        base = (c * NS + s) * EPW

        def emit(i, j3, j2, wait_idx, issue_idx, issue_gather):
            if wait_idx:
                pltpu.make_async_copy(src_hbm.at[pl.ds(0, CH)],
                                      sidx[(j3 + 1) % 3], isem).wait()
                pltpu.make_async_copy(dst_hbm.at[pl.ds(0, CH)],
                                      didx[(j3 + 1) % 3], isem).wait()
            if issue_idx:
                off2 = base + (i + 2) * CH
                pltpu.async_copy(src_hbm.at[pl.ds(off2, CH)],
                                 sidx[(j3 + 2) % 3], isem)
                pltpu.async_copy(dst_hbm.at[pl.ds(off2, CH)],
                                 didx[(j3 + 2) % 3], isem)
            if issue_gather:
                pltpu.async_copy(h_hbm.at[sidx[(j3 + 1) % 3]],
                                 rows[(j2 + 1) % 2], gsem)
            pltpu.make_async_copy(h_hbm.at[sidx[j3]], rows[j2], gsem).wait()
            pltpu.sync_copy(rows[j2], acc.at[didx[j3]], add=True)

        pltpu.sync_copy(src_hbm.at[pl.ds(base, CH)], sidx[0])
        pltpu.sync_copy(dst_hbm.at[pl.ds(base, CH)], didx[0])
        pltpu.async_copy(src_hbm.at[pl.ds(base + CH, CH)], sidx[1], isem)
        pltpu.async_copy(dst_hbm.at[pl.ds(base + CH, CH)], didx[1], isem)
        pltpu.async_copy(h_hbm.at[sidx[0]], rows[0], gsem)

        def blk(b, carry):
            i0 = 6 * b
            for j in range(6):
                emit(i0 + j, j % 3, j % 2, True, True, True)
            return carry

        lax.fori_loop(0, (NFULL - 6) // 6, blk, 0)
        for i in range(NFULL - 6, NFULL):
            emit(i, i % 3, i % 2,
                 wait_idx=(i < NFULL - 1),
                 issue_idx=(i + 2 < NFULL),
                 issue_gather=(i + 1 < NFULL))

        toff = base + NFULL * CH
        pltpu.sync_copy(src_hbm.at[pl.ds(toff, TAIL)], sidx_t)
        pltpu.sync_copy(dst_hbm.at[pl.ds(toff, TAIL)], didx_t)
        pltpu.async_copy(h_hbm.at[sidx_t], rows_t, gsem).wait()
        pltpu.sync_copy(rows_t, acc.at[didx_t], add=True)
        plsc.subcore_barrier()

        @pl.when(s < NS - 1)
        def _():
            pltpu.sync_copy(acc.at[pl.ds(start, RPT)],
                            out_hbm.at[c, pl.ds(start, RPT)])

        @pl.when(s == NS - 1)
        def _():
            pltpu.sync_copy(acc.at[pl.ds(start, RPT_LAST)],
                            out_hbm.at[c, pl.ds(start, RPT_LAST)])

    return scatter_kernel


_SCAT64 = _sc_scatter(64)
_SCAT128 = _sc_scatter(128)


def _leaky(x):
    return jnp.where(x >= 0, x, 0.01 * x)


_BM = 1000
_BC = 2000


def _make_combine(F, n_extra, bias, act):
    def body(*refs):
        refs = list(refs)
        o_ref = refs.pop()
        b_ref = refs.pop() if bias else None
        p_ref = refs.pop(0)
        t = p_ref[0] + p_ref[1]
        for r in refs:
            t = t + r[...]
        if b_ref is not None:
            t = t + b_ref[...]
        if act:
            t = _leaky(t)
        o_ref[...] = t

    in_specs = [pl.BlockSpec((2, _BC, F), lambda i: (0, i, 0))]
    in_specs += [pl.BlockSpec((_BC, F), lambda i: (i, 0))] * n_extra
    if bias:
        in_specs.append(pl.BlockSpec((1, F), lambda i: (0, 0)))
    return pl.pallas_call(
        body,
        grid=(N // _BC,),
        in_specs=in_specs,
        out_specs=pl.BlockSpec((_BC, F), lambda i: (i, 0)),
        out_shape=jax.ShapeDtypeStruct((N, F), jnp.float32),
    )


_COMB64 = _make_combine(64, 0, False, False)
_COMB128 = _make_combine(128, 0, False, False)


def _make_mm4(fin, fout, act):
    def body(h0, h1, h2, h3, w_ref, b_ref, o_ref):
        acc = jnp.dot(h0[...], w_ref[0], preferred_element_type=jnp.float32)
        for k, h in enumerate((h1, h2, h3), start=1):
            acc = acc + jnp.dot(h[...], w_ref[k],
                                preferred_element_type=jnp.float32)
        acc = acc + b_ref[...]
        if act:
            acc = _leaky(acc)
        o_ref[...] = acc

    return pl.pallas_call(
        body,
        grid=(N // _BM,),
        in_specs=[pl.BlockSpec((_BM, fin), lambda i: (i, 0))] * 4
        + [pl.BlockSpec((4, fin, fout), lambda i: (0, 0, 0)),
           pl.BlockSpec((1, fout), lambda i: (0, 0))],
        out_specs=pl.BlockSpec((_BM, fout), lambda i: (i, 0)),
        out_shape=jax.ShapeDtypeStruct((N, fout), jnp.float32),
    )


_MM4_L1 = _make_mm4(128, 64, True)
_MM4_L2 = _make_mm4(64, 128, True)
_MM4_L3 = _make_mm4(128, 256, False)

_BP = 1000


def _pool_body(b_ref, h_ref, o_ref):
    @pl.when(pl.program_id(0) == 0)
    def _init():
        o_ref[...] = jnp.full((G, 256), -jnp.inf, jnp.float32)

    bi = b_ref[0]
    h = h_ref[...]
    lo = jnp.min(bi)
    hi = jnp.max(bi)

    def gbody(g, carry):
        m = bi == g
        v = jnp.max(jnp.where(m, h, -jnp.inf), axis=0, keepdims=True)
        o_ref[pl.ds(g, 1), :] = jnp.maximum(o_ref[pl.ds(g, 1), :], v)
        return carry

    lax.fori_loop(lo, hi + 1, gbody, 0)


_POOL = pl.pallas_call(
    _pool_body,
    grid=(N // _BP,),
    in_specs=[pl.BlockSpec((1, _BP, 1), lambda i: (i, 0, 0)),
              pl.BlockSpec((_BP, 256), lambda i: (i, 0))],
    out_specs=pl.BlockSpec((G, 256), lambda i: (0, 0)),
    out_shape=jax.ShapeDtypeStruct((G, 256), jnp.float32),
)


def _mlp_body(g_ref, w1, b1, w2, b2, w3, b3, o_ref):
    g = g_ref[...]
    g = jnp.where(jnp.isfinite(g), g, 0.0)
    h = jnp.maximum(
        jnp.dot(g, w1[...], preferred_element_type=jnp.float32) + b1[...], 0.0)
    h = jnp.maximum(
        jnp.dot(h, w2[...], preferred_element_type=jnp.float32) + b2[...], 0.0)
    o_ref[...] = jnp.dot(h, w3[...], preferred_element_type=jnp.float32) + b3[...]


_MLP = pl.pallas_call(
    _mlp_body,
    out_shape=jax.ShapeDtypeStruct((G, 4), jnp.float32),
)


def kernel(x, edge_index, batch, W1, b1, W2, b2, W3, b3,
           Wc1, bc1, Wc2, bc2, Wc3, bc3):
    src = edge_index[0]
    dst = edge_index[1]
    z64 = jnp.zeros((RPT_LAST, 64), jnp.float32)
    z128 = jnp.zeros((RPT_LAST, 128), jnp.float32)

    p = _SCAT128(x, src, dst, z128)
    a1 = _COMB128(p)
    p = _SCAT128(a1, src, dst, z128)
    a2 = _COMB128(p)
    p = _SCAT128(a2, src, dst, z128)
    a3 = _COMB128(p)
    h1 = _MM4_L1(x, a1, a2, a3, W1, b1.reshape(1, 64))

    p = _SCAT64(h1, src, dst, z64)
    h2 = _COMB64(p)
    p = _SCAT64(h2, src, dst, z64)
    h3 = _COMB64(p)
    p = _SCAT64(h3, src, dst, z64)
    h4 = _COMB64(p)
    z = _MM4_L2(h1, h2, h3, h4, W2, b2.reshape(1, 128))

    p = _SCAT128(z, src, dst, z128)
    g2 = _COMB128(p)
    p = _SCAT128(g2, src, dst, z128)
    g3 = _COMB128(p)
    p = _SCAT128(g3, src, dst, z128)
    g4 = _COMB128(p)
    hfin = _MM4_L3(z, g2, g3, g4, W3, b3.reshape(1, 256))

    gpool = _POOL(batch.reshape(N // _BP, _BP, 1), hfin)
    return _MLP(gpool, Wc1, bc1.reshape(1, 1024), Wc2, bc2.reshape(1, 512),
                Wc3, bc3.reshape(1, 4))


# speedup vs baseline: 2.2792x; 1.0771x over previous
import functools

import jax
import jax.numpy as jnp
from jax import lax
from jax.experimental import pallas as pl
from jax.experimental.pallas import tpu as pltpu
from jax.experimental.pallas import tpu_sc as plsc

N = 10000
E = 320000
G = 64
NC, NS = 2, 16
NW = NC * NS
EPW = E // NW
CH = 128
NFULL = EPW // CH
TAIL = EPW - NFULL * CH
RPT = 624
RPT_LAST = N - (NS - 1) * RPT


def _sc_scatter(F):
    mesh = plsc.VectorSubcoreMesh(core_axis_name="c", subcore_axis_name="s",
                                  num_cores=NC, num_subcores=NS)
    NB = 4 if F <= 64 else 3

    @functools.partial(
        pl.kernel,
        out_type=jax.ShapeDtypeStruct((NC, N, F), jnp.float32),
        mesh=mesh,
        scratch_types=[
            [pltpu.VMEM((CH,), jnp.int32)] * 4,
            [pltpu.VMEM((CH,), jnp.int32)] * 4,
            [pltpu.VMEM((CH, F), jnp.float32)] * NB,
            pltpu.VMEM((TAIL,), jnp.int32),
            pltpu.VMEM((TAIL,), jnp.int32),
            pltpu.VMEM_SHARED((N, F), jnp.float32),
            pltpu.SemaphoreType.DMA,
            pltpu.SemaphoreType.DMA,
            pltpu.SemaphoreType.DMA,
        ],
        compiler_params=pltpu.CompilerParams(use_tc_tiling_on_sc=False),
    )
    def scatter_kernel(h_hbm, src_hbm, dst_hbm, zeros_hbm, out_hbm,
                       sidx, didx, rows, sidx_t, didx_t, acc,
                       isem, gsem, ssem):
        c = lax.axis_index("c")
        s = lax.axis_index("s")
        start = s * RPT

        @pl.when(s < NS - 1)
        def _():
            pltpu.sync_copy(zeros_hbm.at[pl.ds(0, RPT)],
                            acc.at[pl.ds(start, RPT)])

        @pl.when(s == NS - 1)
        def _():
            pltpu.sync_copy(zeros_hbm, acc.at[pl.ds(start, RPT_LAST)])

        plsc.subcore_barrier()
        base = (c * NS + s) * EPW

        def emit(i, s4, sr, wait_idx, wait_scat, issue_idx, issue_gather,
                 issue_scat=True):
            if wait_idx:
                pltpu.make_async_copy(src_hbm.at[pl.ds(0, CH)],
                                      sidx[(s4 + 1) % 4], isem).wait()
                pltpu.make_async_copy(dst_hbm.at[pl.ds(0, CH)],
                                      didx[(s4 + 1) % 4], isem).wait()
            if wait_scat:
                pltpu.make_async_copy(rows[(sr + NB - 2) % NB],
                                      acc.at[didx[(s4 + 2) % 4]], ssem).wait()
            if issue_idx:
                off2 = base + (i + 2) * CH
                pltpu.async_copy(src_hbm.at[pl.ds(off2, CH)],
                                 sidx[(s4 + 2) % 4], isem)
                pltpu.async_copy(dst_hbm.at[pl.ds(off2, CH)],
                                 didx[(s4 + 2) % 4], isem)
            if issue_gather:
                pltpu.async_copy(h_hbm.at[sidx[(s4 + 1) % 4]],
                                 rows[(sr + 1) % NB], gsem)
            pltpu.make_async_copy(h_hbm.at[sidx[s4]], rows[sr], gsem).wait()
            if issue_scat:
                pltpu.async_copy(rows[sr], acc.at[didx[s4]], ssem, add=True)

        pltpu.sync_copy(src_hbm.at[pl.ds(base, CH)], sidx[0])
        pltpu.sync_copy(dst_hbm.at[pl.ds(base, CH)], didx[0])
        pltpu.async_copy(src_hbm.at[pl.ds(base + CH, CH)], sidx[1], isem)
        pltpu.async_copy(dst_hbm.at[pl.ds(base + CH, CH)], didx[1], isem)
        pltpu.async_copy(h_hbm.at[sidx[0]], rows[0], gsem)
        emit(0, 0, 0, True, False, True, True)
        emit(1, 1, 1, True, False, True, True)

        def blk(b, carry):
            for j in range(12):
                emit(2 + 12 * b + j, (2 + j) % 4, (2 + j) % NB,
                     True, True, True, True)
            return carry

        lax.fori_loop(0, (NFULL - 6) // 12, blk, 0)
        for i in range(NFULL - 4, NFULL):
            emit(i, i % 4, i % NB,
                 wait_idx=(i < NFULL - 1),
                 wait_scat=True,
                 issue_idx=(i + 2 < NFULL),
                 issue_gather=(i + 1 < NFULL))
        pltpu.make_async_copy(rows[(NFULL - 2) % NB],
                              acc.at[didx[(NFULL - 2) % 4]], ssem).wait()
        pltpu.make_async_copy(rows[(NFULL - 1) % NB],
                              acc.at[didx[(NFULL - 1) % 4]], ssem).wait()

        toff = base + NFULL * CH
        pltpu.sync_copy(src_hbm.at[pl.ds(toff, TAIL)], sidx_t)
        pltpu.sync_copy(dst_hbm.at[pl.ds(toff, TAIL)], didx_t)
        tr = rows[0].at[pl.ds(0, TAIL)]
        pltpu.async_copy(h_hbm.at[sidx_t], tr, gsem).wait()
        pltpu.sync_copy(tr, acc.at[didx_t], add=True)
        plsc.subcore_barrier()

        @pl.when(s < NS - 1)
        def _():
            pltpu.sync_copy(acc.at[pl.ds(start, RPT)],
                            out_hbm.at[c, pl.ds(start, RPT)])

        @pl.when(s == NS - 1)
        def _():
            pltpu.sync_copy(acc.at[pl.ds(start, RPT_LAST)],
                            out_hbm.at[c, pl.ds(start, RPT_LAST)])

    return scatter_kernel


_SCAT64 = _sc_scatter(64)
_SCAT128 = _sc_scatter(128)


def _leaky(x):
    return jnp.where(x >= 0, x, 0.01 * x)


_BM = 1000
_BC = 2000


def _make_combine(F, n_extra, bias, act):
    def body(*refs):
        refs = list(refs)
        o_ref = refs.pop()
        b_ref = refs.pop() if bias else None
        p_ref = refs.pop(0)
        t = p_ref[0] + p_ref[1]
        for r in refs:
            t = t + r[...]
        if b_ref is not None:
            t = t + b_ref[...]
        if act:
            t = _leaky(t)
        o_ref[...] = t

    in_specs = [pl.BlockSpec((2, _BC, F), lambda i: (0, i, 0))]
    in_specs += [pl.BlockSpec((_BC, F), lambda i: (i, 0))] * n_extra
    if bias:
        in_specs.append(pl.BlockSpec((1, F), lambda i: (0, 0)))
    return pl.pallas_call(
        body,
        grid=(N // _BC,),
        in_specs=in_specs,
        out_specs=pl.BlockSpec((_BC, F), lambda i: (i, 0)),
        out_shape=jax.ShapeDtypeStruct((N, F), jnp.float32),
    )


_COMB64 = _make_combine(64, 0, False, False)
_COMB128 = _make_combine(128, 0, False, False)


def _make_mm4(fin, fout, act):
    def body(h0, h1, h2, h3, w_ref, b_ref, o_ref):
        acc = jnp.dot(h0[...], w_ref[0], preferred_element_type=jnp.float32)
        for k, h in enumerate((h1, h2, h3), start=1):
            acc = acc + jnp.dot(h[...], w_ref[k],
                                preferred_element_type=jnp.float32)
        acc = acc + b_ref[...]
        if act:
            acc = _leaky(acc)
        o_ref[...] = acc

    return pl.pallas_call(
        body,
        grid=(N // _BM,),
        in_specs=[pl.BlockSpec((_BM, fin), lambda i: (i, 0))] * 4
        + [pl.BlockSpec((4, fin, fout), lambda i: (0, 0, 0)),
           pl.BlockSpec((1, fout), lambda i: (0, 0))],
        out_specs=pl.BlockSpec((_BM, fout), lambda i: (i, 0)),
        out_shape=jax.ShapeDtypeStruct((N, fout), jnp.float32),
    )


_MM4_L1 = _make_mm4(128, 64, True)
_MM4_L2 = _make_mm4(64, 128, True)
_MM4_L3 = _make_mm4(128, 256, False)

_BP = 1000


def _pool_body(b_ref, h_ref, o_ref):
    @pl.when(pl.program_id(0) == 0)
    def _init():
        o_ref[...] = jnp.full((G, 256), -jnp.inf, jnp.float32)

    bi = b_ref[0]
    h = h_ref[...]
    lo = jnp.min(bi)
    hi = jnp.max(bi)

    def gbody(g, carry):
        m = bi == g
        v = jnp.max(jnp.where(m, h, -jnp.inf), axis=0, keepdims=True)
        o_ref[pl.ds(g, 1), :] = jnp.maximum(o_ref[pl.ds(g, 1), :], v)
        return carry

    lax.fori_loop(lo, hi + 1, gbody, 0)


_POOL = pl.pallas_call(
    _pool_body,
    grid=(N // _BP,),
    in_specs=[pl.BlockSpec((1, _BP, 1), lambda i: (i, 0, 0)),
              pl.BlockSpec((_BP, 256), lambda i: (i, 0))],
    out_specs=pl.BlockSpec((G, 256), lambda i: (0, 0)),
    out_shape=jax.ShapeDtypeStruct((G, 256), jnp.float32),
)


def _mlp_body(g_ref, w1, b1, w2, b2, w3, b3, o_ref):
    g = g_ref[...]
    g = jnp.where(jnp.isfinite(g), g, 0.0)
    h = jnp.maximum(
        jnp.dot(g, w1[...], preferred_element_type=jnp.float32) + b1[...], 0.0)
    h = jnp.maximum(
        jnp.dot(h, w2[...], preferred_element_type=jnp.float32) + b2[...], 0.0)
    o_ref[...] = jnp.dot(h, w3[...], preferred_element_type=jnp.float32) + b3[...]


_MLP = pl.pallas_call(
    _mlp_body,
    out_shape=jax.ShapeDtypeStruct((G, 4), jnp.float32),
)


def kernel(x, edge_index, batch, W1, b1, W2, b2, W3, b3,
           Wc1, bc1, Wc2, bc2, Wc3, bc3):
    src = edge_index[0]
    dst = edge_index[1]
    z64 = jnp.zeros((RPT_LAST, 64), jnp.float32)
    z128 = jnp.zeros((RPT_LAST, 128), jnp.float32)

    p = _SCAT128(x, src, dst, z128)
    a1 = _COMB128(p)
    p = _SCAT128(a1, src, dst, z128)
    a2 = _COMB128(p)
    p = _SCAT128(a2, src, dst, z128)
    a3 = _COMB128(p)
    h1 = _MM4_L1(x, a1, a2, a3, W1, b1.reshape(1, 64))

    p = _SCAT64(h1, src, dst, z64)
    h2 = _COMB64(p)
    p = _SCAT64(h2, src, dst, z64)
    h3 = _COMB64(p)
    p = _SCAT64(h3, src, dst, z64)
    h4 = _COMB64(p)
    z = _MM4_L2(h1, h2, h3, h4, W2, b2.reshape(1, 128))

    p = _SCAT128(z, src, dst, z128)
    g2 = _COMB128(p)
    p = _SCAT128(g2, src, dst, z128)
    g3 = _COMB128(p)
    p = _SCAT128(g3, src, dst, z128)
    g4 = _COMB128(p)
    hfin = _MM4_L3(z, g2, g3, g4, W3, b3.reshape(1, 256))

    gpool = _POOL(batch.reshape(N // _BP, _BP, 1), hfin)
    return _MLP(gpool, Wc1, bc1.reshape(1, 1024), Wc2, bc2.reshape(1, 512),
                Wc3, bc3.reshape(1, 4))
